# baseline (speedup 1.0000x reference)
import functools

import jax
import jax.numpy as jnp
from jax import lax
from jax.experimental import pallas as pl
from jax.experimental.pallas import tpu as pltpu
from jax.experimental.pallas import tpu_sc as plsc

N_ROWS = 4096
N_COLS = 200
D = 64
B = N_ROWS * N_COLS
V = 1000000

NC = 2
NS = 16
NW = NC * NS

_mesh = plsc.VectorSubcoreMesh(core_axis_name="c", subcore_axis_name="s")


VC = 512
N_FULL = V // VC
TAIL = V - N_FULL * VC
STEPS1 = (N_FULL + NW - 1) // NW


@functools.partial(
    pl.kernel,
    mesh=_mesh,
    out_type=jax.ShapeDtypeStruct((V // 2, 128), jnp.float32),
    scratch_types=[
        pltpu.VMEM((D, VC), jnp.float32),
        pltpu.VMEM((2, VC // 2, 128), jnp.float32),
        pltpu.VMEM((TAIL, D), jnp.float32),
        pltpu.VMEM((TAIL // 2, 128), jnp.float32),
        pltpu.SemaphoreType.DMA,
        pltpu.SemaphoreType.DMA,
    ],
    compiler_params=pltpu.CompilerParams(needs_layout_passes=False),
)
def _repack_kernel(wt_hbm, tail_hbm, p_hbm, slab, pblk, slab_t, ptail,
                   sem_w0, sem_w1):
    wid = lax.axis_index("s") * NC + lax.axis_index("c")
    iota = lax.iota(jnp.int32, 16)

    def transpose_block(n_rr, s):
        def body(rr, carry):
            for tt in range(8):
                k0 = (16 * tt) % 64
                h = tt // 4
                col = jnp.full((16,), 2 * rr + h, jnp.int32)
                val = plsc.load_gather(slab, [k0 + iota, col])
                pblk[s, rr, pl.ds(16 * tt, 16)] = val
            return carry
        lax.fori_loop(0, n_rr, body, 0)

    def wait_w(s, size):
        @pl.when(s == 0)
        def _():
            pltpu.make_async_copy(pblk.at[0, pl.ds(0, size)],
                                  p_hbm.at[pl.ds(0, size)], sem_w0).wait()
        @pl.when(s != 0)
        def _():
            pltpu.make_async_copy(pblk.at[1, pl.ds(0, size)],
                                  p_hbm.at[pl.ds(0, size)], sem_w1).wait()

    def start_w(s, size, dst0):
        @pl.when(s == 0)
        def _():
            pltpu.make_async_copy(pblk.at[0, pl.ds(0, size)],
                                  p_hbm.at[pl.ds(dst0, size)], sem_w0).start()
        @pl.when(s != 0)
        def _():
            pltpu.make_async_copy(pblk.at[1, pl.ds(0, size)],
                                  p_hbm.at[pl.ds(dst0, size)], sem_w1).start()

    def step(t, carry):
        c = wid + NW * t
        s = lax.rem(t, 2)

        @pl.when(c < N_FULL)
        def _():
            pltpu.sync_copy(wt_hbm.at[:, pl.ds(c * VC, VC)], slab)
            @pl.when(t >= 2)
            def _():
                wait_w(s, VC // 2)
            transpose_block(VC // 2, s)
            start_w(s, VC // 2, c * (VC // 2))

        return carry

    lax.fori_loop(0, STEPS1, step, 0)

    @pl.when(wid == NW - 1)
    def _():
        pltpu.sync_copy(tail_hbm, slab_t)

        def tail_body(rr, carry):
            for tt in range(8):
                k0 = (16 * tt) % 64
                h = tt // 4
                row = jnp.full((16,), 2 * rr + h, jnp.int32)
                val = plsc.load_gather(slab_t, [row, k0 + iota])
                ptail[rr, pl.ds(16 * tt, 16)] = val
            return carry
        lax.fori_loop(0, TAIL // 2, tail_body, 0)
        pltpu.sync_copy(ptail, p_hbm.at[pl.ds(N_FULL * (VC // 2), TAIL // 2)])

    n_mine = (N_FULL - 1 - wid) // NW + 1
    last_t = n_mine - 1
    s_last = lax.rem(last_t, 2)
    wait_w(1 - s_last, VC // 2)
    wait_w(s_last, VC // 2)


IB = 256
N_IBLK = N_ROWS // IB
N_UNITS = N_COLS * N_IBLK
STEPS2 = N_UNITS // NW


@functools.partial(
    pl.kernel,
    mesh=_mesh,
    out_type=jax.ShapeDtypeStruct((N_COLS, D, N_ROWS), jnp.float32),
    scratch_types=[
        pltpu.VMEM((IB,), jnp.int32),
        pltpu.VMEM((IB,), jnp.int32),
        pltpu.VMEM((IB,), jnp.int32),
        pltpu.VMEM((IB, 128), jnp.float32),
        pltpu.VMEM((D, IB), jnp.float32),
        pltpu.SemaphoreType.DMA,
        pltpu.SemaphoreType.DMA,
    ],
    compiler_params=pltpu.CompilerParams(needs_layout_passes=False),
)
def _gather_kernel(xt_hbm, p_hbm, out_hbm, idx_v, idxr, cb, rows_v, oslab,
                   sem_g, sem_o):
    wid = lax.axis_index("s") * NC + lax.axis_index("c")
    iota = lax.iota(jnp.int32, 16)

    def step(t, carry):
        u = wid + NW * t
        j = u // N_IBLK
        i0 = (u % N_IBLK) * IB

        pltpu.sync_copy(xt_hbm.at[j, pl.ds(i0, IB)], idx_v)

        def prep(g, carry2):
            v = idx_v[pl.ds(16 * g, 16)]
            idxr[pl.ds(16 * g, 16)] = lax.shift_right_logical(v, 1)
            cb[pl.ds(16 * g, 16)] = lax.shift_left(lax.bitwise_and(v, 1), 6)
            return carry2
        lax.fori_loop(0, IB // 16, prep, 0)

        pltpu.make_async_copy(p_hbm.at[idxr], rows_v, sem_g).start()
        pltpu.make_async_copy(p_hbm.at[idxr], rows_v, sem_g).wait()

        @pl.when(t >= 1)
        def _():
            pltpu.make_async_copy(oslab, out_hbm.at[0, :, pl.ds(0, IB)],
                                  sem_o).wait()

        def tblock(g, carry2):
            cbv = cb[pl.ds(16 * g, 16)]
            row = 16 * g + iota

            def inner(k, carry3):
                val = plsc.load_gather(rows_v, [row, cbv + k])
                oslab[k, pl.ds(16 * g, 16)] = val
                return carry3
            lax.fori_loop(0, D, inner, 0)
            return carry2
        lax.fori_loop(0, IB // 16, tblock, 0)

        pltpu.make_async_copy(oslab, out_hbm.at[j, :, pl.ds(i0, IB)],
                              sem_o).start()
        return carry

    lax.fori_loop(0, STEPS2, step, 0)
    pltpu.make_async_copy(oslab, out_hbm.at[0, :, pl.ds(0, IB)], sem_o).wait()


def kernel(x, W):
    xt = x.T.astype(jnp.int32)
    p = _repack_kernel(W.T, W[N_FULL * VC:])
    out_t = _gather_kernel(xt, p)
    return jnp.transpose(out_t, (2, 0, 1))


# speedup vs baseline: 1.6730x; 1.6730x over previous
import functools

import jax
import jax.numpy as jnp
from jax import lax
from jax.experimental import pallas as pl
from jax.experimental.pallas import tpu as pltpu
from jax.experimental.pallas import tpu_sc as plsc

N_ROWS = 4096
N_COLS = 200
D = 64
V = 1000000

NC = 2
NS = 16
NW = NC * NS

IB = 256
N_IBLK = N_ROWS // IB
N_UNITS = N_COLS * N_IBLK
STEPS = N_UNITS // NW

_mesh = plsc.VectorSubcoreMesh(core_axis_name="c", subcore_axis_name="s")


@functools.partial(
    pl.kernel,
    mesh=_mesh,
    out_type=jax.ShapeDtypeStruct((N_COLS, D, N_ROWS), jnp.float32),
    scratch_types=[
        pltpu.VMEM((2, IB), jnp.int32),
        pltpu.VMEM((IB,), jnp.int32),
        pltpu.VMEM((IB,), jnp.int32),
        pltpu.VMEM((2, IB), jnp.int32),
        pltpu.VMEM((D, 16), jnp.int32),
        pltpu.VMEM((2, IB, 128), jnp.float32),
        pltpu.VMEM((2, D, IB), jnp.float32),
        pltpu.SemaphoreType.DMA,
        pltpu.SemaphoreType.DMA,
        pltpu.SemaphoreType.DMA,
    ],
    compiler_params=pltpu.CompilerParams(needs_layout_passes=False),
)
def _gather_kernel(xt_hbm, p_hbm, out_hbm, idx_v, idxr0, idxr1, cb, km,
                   rows_v, oslab, sem_i, sem_g, sem_o):
    wid = lax.axis_index("s") * NC + lax.axis_index("c")
    iota = lax.iota(jnp.int32, 16)

    def km_init(e, carry):
        q = e // 16
        d = e % 16
        km[e, :] = 16 * q + lax.bitwise_and(iota + d, 15)
        return carry
    lax.fori_loop(0, D, km_init, 0)

    def unit(t):
        u = wid + NW * t
        return u // N_IBLK, (u % N_IBLK) * IB

    def start_idx(t, s):
        j, i0 = unit(t)
        pltpu.make_async_copy(xt_hbm.at[j, pl.ds(i0, IB)], idx_v.at[s],
                              sem_i).start()

    def prep(t, s):
        pltpu.make_async_copy(xt_hbm.at[0, pl.ds(0, IB)], idx_v.at[s],
                              sem_i).wait()

        def body(g, carry):
            v = idx_v[s, pl.ds(16 * g, 16)]
            r = lax.shift_right_logical(v, 1)
            @pl.when(s == 0)
            def _():
                idxr0[pl.ds(16 * g, 16)] = r
            @pl.when(s != 0)
            def _():
                idxr1[pl.ds(16 * g, 16)] = r
            cb[s, pl.ds(16 * g, 16)] = lax.shift_left(
                lax.bitwise_and(v, 1), 6)
            return carry
        lax.fori_loop(0, IB // 16, body, 0)

    def start_gather(s):
        @pl.when(s == 0)
        def _():
            pltpu.make_async_copy(p_hbm.at[idxr0], rows_v.at[0],
                                  sem_g).start()
        @pl.when(s != 0)
        def _():
            pltpu.make_async_copy(p_hbm.at[idxr1], rows_v.at[1],
                                  sem_g).start()

    def wait_gather(s):
        @pl.when(s == 0)
        def _():
            pltpu.make_async_copy(p_hbm.at[idxr0], rows_v.at[0],
                                  sem_g).wait()
        @pl.when(s != 0)
        def _():
            pltpu.make_async_copy(p_hbm.at[idxr1], rows_v.at[1],
                                  sem_g).wait()

    def transpose(s):
        def gbody(g, carry):
            col = 16 * g + iota
            cbv = cb[s, pl.ds(16 * g, 16)]

            def ebody(e, carry2):
                kv = km[e, :]
                val = plsc.load_gather(rows_v.at[s], [col, cbv + kv])
                plsc.store_scatter(oslab.at[s], [kv, col], val)
                return carry2
            lax.fori_loop(0, D, ebody, 0)
            return carry
        lax.fori_loop(0, IB // 16, gbody, 0)

    def start_out(t, s):
        j, i0 = unit(t)
        pltpu.make_async_copy(oslab.at[s], out_hbm.at[j, :, pl.ds(i0, IB)],
                              sem_o).start()

    def wait_out(s):
        pltpu.make_async_copy(oslab.at[s], out_hbm.at[0, :, pl.ds(0, IB)],
                              sem_o).wait()

    start_idx(0, 0)
    prep(0, 0)
    start_gather(0)
    start_idx(1, 1)

    def step(t, carry):
        s = lax.rem(t, 2)

        @pl.when(t + 1 < STEPS)
        def _():
            @pl.when(t + 2 < STEPS)
            def _():
                start_idx(t + 2, s)
            prep(t + 1, 1 - s)

        wait_gather(s)

        @pl.when(t + 1 < STEPS)
        def _():
            start_gather(1 - s)

        @pl.when(t >= 2)
        def _():
            wait_out(s)
        transpose(s)
        start_out(t, s)
        return carry

    lax.fori_loop(0, STEPS, step, 0)
    wait_out(lax.rem(STEPS - 2, 2))
    wait_out(lax.rem(STEPS - 1, 2))


def kernel(x, W):
    xt = x.T.astype(jnp.int32)
    p = W.reshape(V // 2, 128)
    out_t = _gather_kernel(xt, p)
    return jnp.transpose(out_t, (2, 0, 1))


# speedup vs baseline: 1.8357x; 1.0973x over previous
import functools

import jax
import jax.numpy as jnp
from jax import lax
from jax.experimental import pallas as pl
from jax.experimental.pallas import tpu as pltpu
from jax.experimental.pallas import tpu_sc as plsc

N_ROWS = 4096
N_COLS = 200
D = 64
V = 1000000

NC = 2
NS = 16
NW = NC * NS

IB = 256
N_IBLK = N_ROWS // IB
N_UNITS = N_COLS * N_IBLK
STEPS = N_UNITS // NW

_mesh = plsc.VectorSubcoreMesh(core_axis_name="c", subcore_axis_name="s")


@functools.partial(
    pl.kernel,
    mesh=_mesh,
    out_type=jax.ShapeDtypeStruct((N_COLS, D, N_ROWS), jnp.float32),
    scratch_types=[
        pltpu.VMEM((2, IB), jnp.int32),
        pltpu.VMEM((IB,), jnp.int32),
        pltpu.VMEM((IB,), jnp.int32),
        pltpu.VMEM((2, IB), jnp.int32),
        pltpu.VMEM((D, 16), jnp.int32),
        pltpu.VMEM((2, IB, 128), jnp.float32),
        pltpu.VMEM((2, D, IB), jnp.float32),
        pltpu.SemaphoreType.DMA,
        pltpu.SemaphoreType.DMA,
        pltpu.SemaphoreType.DMA,
    ],
    compiler_params=pltpu.CompilerParams(needs_layout_passes=False),
)
def _gather_kernel(xt_hbm, p_hbm, out_hbm, idx_v, idxr0, idxr1, cb, km,
                   rows_v, oslab, sem_i, sem_g, sem_o):
    wid = lax.axis_index("s") * NC + lax.axis_index("c")
    iota = lax.iota(jnp.int32, 16)

    def km_init(e, carry):
        q = e // 16
        d = e % 16
        km[e, :] = 16 * q + lax.bitwise_and(iota + d, 15)
        return carry
    lax.fori_loop(0, D, km_init, 0)

    def unit(t):
        u = wid + NW * t
        return u // N_IBLK, (u % N_IBLK) * IB

    def start_idx(t, s):
        j, i0 = unit(t)
        pltpu.make_async_copy(xt_hbm.at[j, pl.ds(i0, IB)], idx_v.at[s],
                              sem_i).start()

    def prep(t, s):
        pltpu.make_async_copy(xt_hbm.at[0, pl.ds(0, IB)], idx_v.at[s],
                              sem_i).wait()

        @pl.when(s == 0)
        def _():
            for g in range(IB // 16):
                v = idx_v[0, pl.ds(16 * g, 16)]
                idxr0[pl.ds(16 * g, 16)] = lax.shift_right_logical(v, 1)
                cb[0, pl.ds(16 * g, 16)] = lax.shift_left(
                    lax.bitwise_and(v, 1), 6)

        @pl.when(s != 0)
        def _():
            for g in range(IB // 16):
                v = idx_v[1, pl.ds(16 * g, 16)]
                idxr1[pl.ds(16 * g, 16)] = lax.shift_right_logical(v, 1)
                cb[1, pl.ds(16 * g, 16)] = lax.shift_left(
                    lax.bitwise_and(v, 1), 6)

    def start_gather(s):
        @pl.when(s == 0)
        def _():
            pltpu.make_async_copy(p_hbm.at[idxr0], rows_v.at[0],
                                  sem_g).start()
        @pl.when(s != 0)
        def _():
            pltpu.make_async_copy(p_hbm.at[idxr1], rows_v.at[1],
                                  sem_g).start()

    def wait_gather(s):
        @pl.when(s == 0)
        def _():
            pltpu.make_async_copy(p_hbm.at[idxr0], rows_v.at[0],
                                  sem_g).wait()
        @pl.when(s != 0)
        def _():
            pltpu.make_async_copy(p_hbm.at[idxr1], rows_v.at[1],
                                  sem_g).wait()

    def transpose(s):
        def gbody(g, carry):
            col = 16 * g + iota
            cbv = cb[s, pl.ds(16 * g, 16)]
            for e in range(D):
                kv = km[e, :]
                val = plsc.load_gather(rows_v.at[s], [col, cbv + kv])
                plsc.store_scatter(oslab.at[s], [kv, col], val)
            return carry
        lax.fori_loop(0, IB // 16, gbody, 0)

    def start_out(t, s):
        j, i0 = unit(t)
        pltpu.make_async_copy(oslab.at[s], out_hbm.at[j, :, pl.ds(i0, IB)],
                              sem_o).start()

    def wait_out(s):
        pltpu.make_async_copy(oslab.at[s], out_hbm.at[0, :, pl.ds(0, IB)],
                              sem_o).wait()

    start_idx(0, 0)
    prep(0, 0)
    start_gather(0)
    start_idx(1, 1)

    def step(t, carry):
        s = lax.rem(t, 2)

        @pl.when(t + 1 < STEPS)
        def _():
            @pl.when(t + 2 < STEPS)
            def _():
                start_idx(t + 2, s)
            prep(t + 1, 1 - s)

        wait_gather(s)

        @pl.when(t + 1 < STEPS)
        def _():
            start_gather(1 - s)

        @pl.when(t >= 2)
        def _():
            wait_out(s)
        transpose(s)
        start_out(t, s)
        return carry

    lax.fori_loop(0, STEPS, step, 0)
    wait_out(lax.rem(STEPS - 2, 2))
    wait_out(lax.rem(STEPS - 1, 2))


def kernel(x, W):
    xt = x.T.astype(jnp.int32)
    p = W.reshape(V // 2, 128)
    out_t = _gather_kernel(xt, p)
    return jnp.transpose(out_t, (2, 0, 1))


# speedup vs baseline: 2.4633x; 1.3419x over previous
import functools

import jax
import jax.numpy as jnp
from jax import lax
from jax.experimental import pallas as pl
from jax.experimental.pallas import tpu as pltpu
from jax.experimental.pallas import tpu_sc as plsc

N_ROWS = 4096
N_COLS = 200
D = 64
B = N_ROWS * N_COLS
V = 1000000

NC = 2
NS = 16
NW = NC * NS
B_PER_W = B // NW
C = 256
STEPS = B_PER_W // C

_mesh = plsc.VectorSubcoreMesh(core_axis_name="c", subcore_axis_name="s")


@functools.partial(
    pl.kernel,
    mesh=_mesh,
    out_type=jax.ShapeDtypeStruct((B, D), jnp.float32),
    scratch_types=[
        pltpu.VMEM((B_PER_W,), jnp.int32),
        pltpu.VMEM((C,), jnp.int32),
        pltpu.VMEM((C,), jnp.int32),
        pltpu.VMEM((2, C), jnp.int32),
        pltpu.VMEM((2, C, 128), jnp.float32),
        pltpu.VMEM((C, D), jnp.float32),
        pltpu.SemaphoreType.DMA,
        pltpu.SemaphoreType.DMA,
    ],
    compiler_params=pltpu.CompilerParams(needs_layout_passes=False),
)
def _gather_kernel(idx_hbm, p_hbm, out_hbm, idx_full, idxr0, idxr1, cbs,
                   rows, wbuf, sem_g, sem_o):
    wid = lax.axis_index("s") * NC + lax.axis_index("c")
    base = wid * B_PER_W

    pltpu.sync_copy(idx_hbm.at[pl.ds(base, B_PER_W)], idx_full)

    def prep(t, s):
        @pl.when(s == 0)
        def _():
            for g in range(C // 16):
                v = idx_full[pl.ds(t * C + 16 * g, 16)]
                idxr0[pl.ds(16 * g, 16)] = lax.shift_right_logical(v, 1)
                cbs[0, pl.ds(16 * g, 16)] = lax.shift_left(
                    lax.bitwise_and(v, 1), 6)
        @pl.when(s != 0)
        def _():
            for g in range(C // 16):
                v = idx_full[pl.ds(t * C + 16 * g, 16)]
                idxr1[pl.ds(16 * g, 16)] = lax.shift_right_logical(v, 1)
                cbs[1, pl.ds(16 * g, 16)] = lax.shift_left(
                    lax.bitwise_and(v, 1), 6)

    def start_gather(s):
        @pl.when(s == 0)
        def _():
            pltpu.make_async_copy(p_hbm.at[idxr0], rows.at[0], sem_g).start()
        @pl.when(s != 0)
        def _():
            pltpu.make_async_copy(p_hbm.at[idxr1], rows.at[1], sem_g).start()

    def wait_gather(s):
        pltpu.make_async_copy(p_hbm.at[idxr0], rows.at[0], sem_g).wait()

    def select(s):
        def body(g, carry):
            cbv = cbs[s, pl.ds(16 * g, 16)]
            for r in range(16):
                i = 16 * g + r
                cbi = cbv[r]
                for q in range(D // 16):
                    lo = rows[s, i, pl.ds(16 * q, 16)]
                    hi = rows[s, i, pl.ds(64 + 16 * q, 16)]
                    wbuf[i, pl.ds(16 * q, 16)] = jnp.where(cbi != 0, hi, lo)
            return carry
        lax.fori_loop(0, C // 16, body, 0)

    def wait_out():
        pltpu.make_async_copy(wbuf, out_hbm.at[pl.ds(0, C)], sem_o).wait()

    prep(0, 0)
    start_gather(0)
    prep(1, 1)

    def step(t, carry):
        s = lax.rem(t, 2)

        @pl.when(t + 1 < STEPS)
        def _():
            start_gather(1 - s)
        wait_gather(s)

        @pl.when(t >= 1)
        def _():
            wait_out()
        select(s)
        pltpu.make_async_copy(wbuf, out_hbm.at[pl.ds(base + t * C, C)],
                              sem_o).start()

        @pl.when(t + 2 < STEPS)
        def _():
            prep(t + 2, s)
        return carry

    lax.fori_loop(0, STEPS, step, 0)
    wait_out()


def kernel(x, W):
    idx = x.reshape(-1).astype(jnp.int32)
    p = W.reshape(V // 2, 128)
    out = _gather_kernel(idx, p)
    return out.reshape(N_ROWS, N_COLS, D)
